# baseline (speedup 1.0000x reference)
import functools

import jax
import jax.numpy as jnp
from jax import lax
from jax.experimental import pallas as pl
from jax.experimental.pallas import tpu as pltpu
from jax.experimental.pallas import tpu_sc as plsc

HIDDEN = 64
OUT = 64
NW = 32
CHUNK = 128


def _gather_sc(idx_flat, emb):
    B = idx_flat.shape[0]
    rows_per_w = B // NW
    nsteps = rows_per_w // CHUNK
    mesh = plsc.VectorSubcoreMesh(core_axis_name="c", subcore_axis_name="s")

    @functools.partial(
        pl.kernel,
        out_type=jax.ShapeDtypeStruct((B, HIDDEN), jnp.float32),
        mesh=mesh,
        scratch_types=[
            pltpu.VMEM((CHUNK,), jnp.int32),
            pltpu.VMEM((CHUNK, HIDDEN), jnp.float32),
            pltpu.SemaphoreType.DMA,
        ],
        compiler_params=pltpu.CompilerParams(use_tc_tiling_on_sc=False),
    )
    def gather_kernel(idx_hbm, emb_hbm, out_hbm, idx_v, rows_v, sem):
        wid = lax.axis_index("s") * 2 + lax.axis_index("c")
        base = wid * rows_per_w

        def step(i, carry):
            start = base + i * CHUNK
            pltpu.sync_copy(idx_hbm.at[pl.ds(start, CHUNK)], idx_v)
            pltpu.async_copy(emb_hbm.at[idx_v], rows_v, sem).wait()
            pltpu.sync_copy(rows_v, out_hbm.at[pl.ds(start, CHUNK)])
            return carry

        lax.fori_loop(0, nsteps, step, 0)

    return gather_kernel(idx_flat, emb)


def _relu_linear_tc(g, W, b8):
    B = g.shape[0]
    BLK = 2048

    def body(g_ref, w_ref, b_ref, o_ref):
        x = jnp.maximum(g_ref[...], 0.0)
        y = lax.dot_general(x, w_ref[...], (((1,), (1,)), ((), ())),
                            preferred_element_type=jnp.float32)
        o_ref[...] = y + b_ref[0:1, :]

    return pl.pallas_call(
        body,
        grid=(B // BLK,),
        in_specs=[
            pl.BlockSpec((BLK, HIDDEN), lambda i: (i, 0)),
            pl.BlockSpec((OUT, HIDDEN), lambda i: (0, 0)),
            pl.BlockSpec((8, OUT), lambda i: (0, 0)),
        ],
        out_specs=pl.BlockSpec((BLK, OUT), lambda i: (i, 0)),
        out_shape=jax.ShapeDtypeStruct((B, OUT), jnp.float32),
    )(g, W, b8)


def kernel(X, emb, W, b):
    batch, hist = X.shape
    idx = X.reshape(-1).astype(jnp.int32)
    g = _gather_sc(idx, emb)
    b8 = jnp.broadcast_to(b[None, :], (8, OUT))
    y = _relu_linear_tc(g, W, b8)
    return y.reshape(batch, hist, OUT)


# speedup vs baseline: 1.1059x; 1.1059x over previous
import functools

import jax
import jax.numpy as jnp
from jax import lax
from jax.experimental import pallas as pl
from jax.experimental.pallas import tpu as pltpu
from jax.experimental.pallas import tpu_sc as plsc

HIDDEN = 64
OUT = 64
NW = 32
CHUNK = 128


K = 8


def _gather_sc(idx_flat, emb):
    B = idx_flat.shape[0]
    rows_per_w = B // NW
    rnd_rows = K * CHUNK
    rounds = rows_per_w // rnd_rows
    mesh = plsc.VectorSubcoreMesh(core_axis_name="c", subcore_axis_name="s")

    @functools.partial(
        pl.kernel,
        out_type=jax.ShapeDtypeStruct((B, HIDDEN), jnp.float32),
        mesh=mesh,
        scratch_types=[
            pltpu.VMEM((2, K, CHUNK), jnp.int32),
            pltpu.VMEM((K, CHUNK, HIDDEN), jnp.float32),
            pltpu.SemaphoreType.DMA((2,)),
            pltpu.SemaphoreType.DMA,
            pltpu.SemaphoreType.DMA,
        ],
        compiler_params=pltpu.CompilerParams(use_tc_tiling_on_sc=False),
    )
    def gather_kernel(idx_hbm, emb_hbm, out_hbm, idx_v, rows_v, idx_sem,
                      g_sem, o_sem):
        wid = lax.axis_index("s") * 2 + lax.axis_index("c")
        base = wid * rows_per_w
        ibase = wid * (rows_per_w // CHUNK)

        def idx_src(r):
            return idx_hbm.at[pl.ds(ibase + r * K, K)]

        def out_dst(r, j):
            return out_hbm.at[pl.ds(base + r * rnd_rows + j * CHUNK, CHUNK)]

        pltpu.async_copy(idx_src(0), idx_v.at[0], idx_sem.at[0])

        def round_body(r, carry):
            s = lax.rem(r, 2)

            @pl.when(r + 1 < rounds)
            def _():
                pltpu.async_copy(idx_src(r + 1), idx_v.at[1 - s],
                                 idx_sem.at[1 - s])

            pltpu.make_async_copy(idx_src(r), idx_v.at[s],
                                  idx_sem.at[s]).wait()

            @pl.when(r > 0)
            def _():
                for j in range(K):
                    pltpu.make_async_copy(rows_v.at[j], out_dst(r - 1, j),
                                          o_sem).wait()

            for j in range(K):
                pltpu.async_copy(emb_hbm.at[idx_v.at[s].at[j]], rows_v.at[j],
                                 g_sem)
            for j in range(K):
                pltpu.make_async_copy(emb_hbm.at[idx_v.at[s].at[j]],
                                      rows_v.at[j], g_sem).wait()

            for j in range(K):
                pltpu.async_copy(rows_v.at[j], out_dst(r, j), o_sem)
            return carry

        lax.fori_loop(0, rounds, round_body, 0)

        for j in range(K):
            pltpu.make_async_copy(rows_v.at[j], out_dst(rounds - 1, j),
                                  o_sem).wait()

    return gather_kernel(idx_flat.reshape(-1, CHUNK), emb)


def _relu_linear_tc(g, W, b8):
    B = g.shape[0]
    BLK = 2048

    def body(g_ref, w_ref, b_ref, o_ref):
        x = jnp.maximum(g_ref[...], 0.0)
        y = lax.dot_general(x, w_ref[...], (((1,), (1,)), ((), ())),
                            preferred_element_type=jnp.float32)
        o_ref[...] = y + b_ref[0:1, :]

    return pl.pallas_call(
        body,
        grid=(B // BLK,),
        in_specs=[
            pl.BlockSpec((BLK, HIDDEN), lambda i: (i, 0)),
            pl.BlockSpec((OUT, HIDDEN), lambda i: (0, 0)),
            pl.BlockSpec((8, OUT), lambda i: (0, 0)),
        ],
        out_specs=pl.BlockSpec((BLK, OUT), lambda i: (i, 0)),
        out_shape=jax.ShapeDtypeStruct((B, OUT), jnp.float32),
    )(g, W, b8)


def kernel(X, emb, W, b):
    batch, hist = X.shape
    idx = X.reshape(-1).astype(jnp.int32)
    g = _gather_sc(idx, emb)
    b8 = jnp.broadcast_to(b[None, :], (8, OUT))
    y = _relu_linear_tc(g, W, b8)
    return y.reshape(batch, hist, OUT)


# speedup vs baseline: 1.2569x; 1.1365x over previous
import functools

import jax
import jax.numpy as jnp
from jax import lax
from jax.experimental import pallas as pl
from jax.experimental.pallas import tpu as pltpu
from jax.experimental.pallas import tpu_sc as plsc

HIDDEN = 64
OUT = 64
NW = 32
CHUNK = 128
K = 4


def _dup_tc(emb):
    M = emb.shape[0]
    BLK = 10000

    def body(e_ref, o_ref):
        e = e_ref[...]
        o_ref[...] = jnp.concatenate([e, e], axis=1)

    return pl.pallas_call(
        body,
        grid=(M // BLK,),
        in_specs=[pl.BlockSpec((BLK, HIDDEN), lambda i: (i, 0))],
        out_specs=pl.BlockSpec((BLK, 2 * HIDDEN), lambda i: (i, 0)),
        out_shape=jax.ShapeDtypeStruct((M, 2 * HIDDEN), jnp.float32),
    )(emb)


def _gather_sc(idx2d, table):
    B = idx2d.shape[0] * idx2d.shape[1]
    rows_per_w = B // NW
    rnd_rows = K * CHUNK
    rounds = rows_per_w // rnd_rows
    mesh = plsc.VectorSubcoreMesh(core_axis_name="c", subcore_axis_name="s")

    @functools.partial(
        pl.kernel,
        out_type=jax.ShapeDtypeStruct((B, 2 * HIDDEN), jnp.float32),
        mesh=mesh,
        scratch_types=[
            pltpu.VMEM((2, K, CHUNK), jnp.int32),
            pltpu.VMEM((rnd_rows, 2 * HIDDEN), jnp.float32),
            pltpu.SemaphoreType.DMA((2,)),
            pltpu.SemaphoreType.DMA,
            pltpu.SemaphoreType.DMA,
        ],
        compiler_params=pltpu.CompilerParams(use_tc_tiling_on_sc=True),
    )
    def gather_kernel(idx_hbm, tab_hbm, out_hbm, idx_v, rows_v, idx_sem,
                      g_sem, o_sem):
        wid = lax.axis_index("s") * 2 + lax.axis_index("c")
        base = wid * rows_per_w
        ibase = wid * (rows_per_w // CHUNK)

        def idx_src(r):
            return idx_hbm.at[pl.ds(ibase + r * K, K)]

        def out_copy(r):
            return pltpu.make_async_copy(
                rows_v,
                out_hbm.at[pl.ds(base + r * rnd_rows, rnd_rows)],
                o_sem)

        pltpu.async_copy(idx_src(0), idx_v.at[0], idx_sem.at[0])

        def round_body(r, carry):
            s = lax.rem(r, 2)

            @pl.when(r + 1 < rounds)
            def _():
                pltpu.async_copy(idx_src(r + 1), idx_v.at[1 - s],
                                 idx_sem.at[1 - s])

            pltpu.make_async_copy(idx_src(r), idx_v.at[s],
                                  idx_sem.at[s]).wait()

            @pl.when(r > 0)
            def _():
                out_copy(r - 1).wait()

            for j in range(K):
                pltpu.async_copy(tab_hbm.at[idx_v.at[s].at[j]],
                                 rows_v.at[pl.ds(j * CHUNK, CHUNK)], g_sem)
            for j in range(K):
                pltpu.make_async_copy(tab_hbm.at[idx_v.at[s].at[j]],
                                      rows_v.at[pl.ds(j * CHUNK, CHUNK)],
                                      g_sem).wait()

            out_copy(r).start()
            return carry

        lax.fori_loop(0, rounds, round_body, 0)
        out_copy(rounds - 1).wait()

    return gather_kernel(idx2d, table)


def _linear_tc(rows, W, b8):
    B = rows.shape[0]
    BLK = 8192

    def body(r_ref, w_ref, b_ref, o_ref):
        x = jnp.maximum(r_ref[:, :HIDDEN], 0.0)
        y = lax.dot_general(x, w_ref[...], (((1,), (1,)), ((), ())),
                            preferred_element_type=jnp.float32)
        o_ref[...] = y + b_ref[0:1, :]

    return pl.pallas_call(
        body,
        grid=(B // BLK,),
        in_specs=[
            pl.BlockSpec((BLK, 2 * HIDDEN), lambda i: (i, 0)),
            pl.BlockSpec((OUT, HIDDEN), lambda i: (0, 0)),
            pl.BlockSpec((8, OUT), lambda i: (0, 0)),
        ],
        out_specs=pl.BlockSpec((BLK, OUT), lambda i: (i, 0)),
        out_shape=jax.ShapeDtypeStruct((B, OUT), jnp.float32),
    )(rows, W, b8)


def kernel(X, emb, W, b):
    batch, hist = X.shape
    idx2d = X.reshape(-1, CHUNK).astype(jnp.int32)
    b8 = jnp.broadcast_to(b[None, :], (8, OUT))
    table = _dup_tc(emb)
    rows = _gather_sc(idx2d, table)
    y = _linear_tc(rows, W, b8)
    return y.reshape(batch, hist, OUT)


# speedup vs baseline: 1.7537x; 1.3952x over previous
import functools

import jax
import jax.numpy as jnp
from jax import lax
from jax.experimental import pallas as pl
from jax.experimental.pallas import tpu as pltpu
from jax.experimental.pallas import tpu_sc as plsc

HIDDEN = 64
OUT = 64
NW = 32
CHUNK = 128
K = 4


def _transform_tc(embT, W, b8):
    M = embT.shape[1]
    BLK = 4096

    def body(e_ref, w_ref, b_ref, o_ref):
        x = jnp.maximum(e_ref[...], 0.0)
        y = lax.dot_general(x, w_ref[...], (((0,), (1,)), ((), ())),
                            preferred_element_type=jnp.float32)
        y = y + b_ref[0:1, :]
        o_ref[...] = jnp.concatenate([y, y], axis=1)

    return pl.pallas_call(
        body,
        grid=(pl.cdiv(M, BLK),),
        in_specs=[
            pl.BlockSpec((HIDDEN, BLK), lambda i: (0, i)),
            pl.BlockSpec((OUT, HIDDEN), lambda i: (0, 0)),
            pl.BlockSpec((8, OUT), lambda i: (0, 0)),
        ],
        out_specs=pl.BlockSpec((BLK, 2 * OUT), lambda i: (i, 0)),
        out_shape=jax.ShapeDtypeStruct((M, 2 * OUT), jnp.float32),
    )(embT, W, b8)


def _gather_sc(idx2d, table):
    B = idx2d.shape[0] * idx2d.shape[1]
    rows_per_w = B // NW
    rnd_rows = K * CHUNK
    rounds = rows_per_w // rnd_rows
    mesh = plsc.VectorSubcoreMesh(core_axis_name="c", subcore_axis_name="s")

    @functools.partial(
        pl.kernel,
        out_type=jax.ShapeDtypeStruct((B, 2 * OUT), jnp.float32),
        mesh=mesh,
        scratch_types=[
            pltpu.VMEM((2, K, CHUNK), jnp.int32),
            pltpu.VMEM((rnd_rows, 2 * OUT), jnp.float32),
            pltpu.SemaphoreType.DMA((2,)),
            pltpu.SemaphoreType.DMA,
            pltpu.SemaphoreType.DMA,
        ],
        compiler_params=pltpu.CompilerParams(use_tc_tiling_on_sc=True),
    )
    def gather_kernel(idx_hbm, tab_hbm, out_hbm, idx_v, rows_v, idx_sem,
                      g_sem, o_sem):
        wid = lax.axis_index("s") * 2 + lax.axis_index("c")
        base = wid * rows_per_w
        ibase = wid * (rows_per_w // CHUNK)

        def idx_src(r):
            return idx_hbm.at[pl.ds(ibase + r * K, K)]

        def out_copy(r):
            return pltpu.make_async_copy(
                rows_v,
                out_hbm.at[pl.ds(base + r * rnd_rows, rnd_rows)],
                o_sem)

        pltpu.async_copy(idx_src(0), idx_v.at[0], idx_sem.at[0])

        def round_body(r, carry):
            s = lax.rem(r, 2)

            @pl.when(r + 1 < rounds)
            def _():
                pltpu.async_copy(idx_src(r + 1), idx_v.at[1 - s],
                                 idx_sem.at[1 - s])

            pltpu.make_async_copy(idx_src(r), idx_v.at[s],
                                  idx_sem.at[s]).wait()

            @pl.when(r > 0)
            def _():
                out_copy(r - 1).wait()

            for j in range(K):
                pltpu.async_copy(tab_hbm.at[idx_v.at[s].at[j]],
                                 rows_v.at[pl.ds(j * CHUNK, CHUNK)], g_sem)
            for j in range(K):
                pltpu.make_async_copy(tab_hbm.at[idx_v.at[s].at[j]],
                                      rows_v.at[pl.ds(j * CHUNK, CHUNK)],
                                      g_sem).wait()

            out_copy(r).start()
            return carry

        lax.fori_loop(0, rounds, round_body, 0)
        out_copy(rounds - 1).wait()

    return gather_kernel(idx2d, table)


def kernel(X, emb, W, b):
    batch, hist = X.shape
    idx2d = X.reshape(-1, CHUNK).astype(jnp.int32)
    b8 = jnp.broadcast_to(b[None, :], (8, OUT))
    table = _transform_tc(emb.T, W, b8)
    rows = _gather_sc(idx2d, table)
    return rows[:, :OUT].reshape(batch, hist, OUT)


# speedup vs baseline: 2.1160x; 1.2066x over previous
import functools

import jax
import jax.numpy as jnp
from jax import lax
from jax.experimental import pallas as pl
from jax.experimental.pallas import tpu as pltpu
from jax.experimental.pallas import tpu_sc as plsc

HIDDEN = 64
OUT = 64
NW = 32
CHUNK = 128
K = 4


def _transform_tc(embT, W, b8):
    M = embT.shape[1]
    BLK = 4096

    def body(e_ref, w_ref, b_ref, o_ref):
        x = jnp.maximum(e_ref[...], 0.0)
        y = lax.dot_general(x, w_ref[...], (((0,), (1,)), ((), ())),
                            preferred_element_type=jnp.float32)
        y = y + b_ref[0:1, :]
        o_ref[...] = jnp.concatenate([y, y], axis=1)

    return pl.pallas_call(
        body,
        grid=(pl.cdiv(M, BLK),),
        in_specs=[
            pl.BlockSpec((HIDDEN, BLK), lambda i: (0, i)),
            pl.BlockSpec((OUT, HIDDEN), lambda i: (0, 0)),
            pl.BlockSpec((8, OUT), lambda i: (0, 0)),
        ],
        out_specs=pl.BlockSpec((BLK, 2 * OUT), lambda i: (i, 0)),
        out_shape=jax.ShapeDtypeStruct((M, 2 * OUT), jnp.float32),
    )(embT, W, b8)


def _gather_sc(idx2d, table):
    B = idx2d.shape[0] * idx2d.shape[1]
    rows_per_w = B // NW
    rnd_rows = K * CHUNK
    rounds = rows_per_w // rnd_rows
    mesh = plsc.VectorSubcoreMesh(core_axis_name="c", subcore_axis_name="s")

    @functools.partial(
        pl.kernel,
        out_type=jax.ShapeDtypeStruct((B, 2 * OUT), jnp.float32),
        mesh=mesh,
        scratch_types=[
            pltpu.VMEM((2, K, CHUNK), jnp.int32),
            pltpu.VMEM((rnd_rows, 2 * OUT), jnp.float32),
            pltpu.SemaphoreType.DMA((2,)),
            pltpu.SemaphoreType.DMA,
            pltpu.SemaphoreType.DMA,
        ],
        compiler_params=pltpu.CompilerParams(use_tc_tiling_on_sc=True),
    )
    def gather_kernel(idx_hbm, tab_hbm, out_hbm, idx_v, rows_v, idx_sem,
                      g_sem, o_sem):
        wid = lax.axis_index("s") * 2 + lax.axis_index("c")
        base = wid * rows_per_w
        ibase = wid * (rows_per_w // CHUNK)

        def idx_src(r):
            return idx_hbm.at[pl.ds(ibase + r * K, K)]

        def out_copy(r):
            return pltpu.make_async_copy(
                rows_v,
                out_hbm.at[pl.ds(base + r * rnd_rows, rnd_rows)],
                o_sem)

        pltpu.async_copy(idx_src(0), idx_v.at[0], idx_sem.at[0])

        def round_body(r, carry):
            s = lax.rem(r, 2)

            @pl.when(r + 1 < rounds)
            def _():
                pltpu.async_copy(idx_src(r + 1), idx_v.at[1 - s],
                                 idx_sem.at[1 - s])

            pltpu.make_async_copy(idx_src(r), idx_v.at[s],
                                  idx_sem.at[s]).wait()

            @pl.when(r > 0)
            def _():
                out_copy(r - 1).wait()

            for j in range(K):
                pltpu.async_copy(tab_hbm.at[idx_v.at[s].at[j]],
                                 rows_v.at[pl.ds(j * CHUNK, CHUNK)], g_sem)
            for j in range(K):
                pltpu.make_async_copy(tab_hbm.at[idx_v.at[s].at[j]],
                                      rows_v.at[pl.ds(j * CHUNK, CHUNK)],
                                      g_sem).wait()

            out_copy(r).start()
            return carry

        lax.fori_loop(0, rounds, round_body, 0)
        out_copy(rounds - 1).wait()

    return gather_kernel(idx2d, table)


def _out_transpose_tc(rows, batch, hist):
    BLKB = 2048

    def body(r_ref, o_ref):
        o_ref[0] = r_ref[:, :OUT].T

    return pl.pallas_call(
        body,
        grid=(hist, batch // BLKB),
        in_specs=[
            pl.BlockSpec((BLKB, 2 * OUT),
                         lambda h, j, nb=batch // BLKB: (h * nb + j, 0)),
        ],
        out_specs=pl.BlockSpec((1, OUT, BLKB), lambda h, j: (h, 0, j)),
        out_shape=jax.ShapeDtypeStruct((hist, OUT, batch), jnp.float32),
    )(rows)


def kernel(X, emb, W, b):
    batch, hist = X.shape
    idx2d = X.T.reshape(-1, CHUNK).astype(jnp.int32)
    b8 = jnp.broadcast_to(b[None, :], (8, OUT))
    table = _transform_tc(emb.T, W, b8)
    rows = _gather_sc(idx2d, table)
    return _out_transpose_tc(rows, batch, hist).transpose(2, 0, 1)


# speedup vs baseline: 2.3251x; 1.0988x over previous
import functools

import jax
import jax.numpy as jnp
from jax import lax
from jax.experimental import pallas as pl
from jax.experimental.pallas import tpu as pltpu
from jax.experimental.pallas import tpu_sc as plsc

HIDDEN = 64
OUT = 64
NW = 32
CHUNK = 128
K = 4


def _transform_tc(embT, W, b8):
    M = embT.shape[1]
    BLK = 4096

    def body(e_ref, w_ref, b_ref, o_ref):
        x = jnp.maximum(e_ref[...], 0.0)
        y = lax.dot_general(x, w_ref[...], (((0,), (1,)), ((), ())),
                            preferred_element_type=jnp.float32)
        y = y + b_ref[0:1, :]
        o_ref[...] = jnp.concatenate([y, y], axis=1)

    return pl.pallas_call(
        body,
        grid=(pl.cdiv(M, BLK),),
        in_specs=[
            pl.BlockSpec((HIDDEN, BLK), lambda i: (0, i)),
            pl.BlockSpec((OUT, HIDDEN), lambda i: (0, 0)),
            pl.BlockSpec((8, OUT), lambda i: (0, 0)),
        ],
        out_specs=pl.BlockSpec((BLK, 2 * OUT), lambda i: (i, 0)),
        out_shape=jax.ShapeDtypeStruct((M, 2 * OUT), jnp.float32),
    )(embT, W, b8)


def _gather_sc(idx2d, table, nrows, row0):
    B = nrows
    rows_per_w = B // NW
    rnd_rows = K * CHUNK
    rounds = rows_per_w // rnd_rows
    mesh = plsc.VectorSubcoreMesh(core_axis_name="c", subcore_axis_name="s")

    @functools.partial(
        pl.kernel,
        out_type=jax.ShapeDtypeStruct((B, 2 * OUT), jnp.float32),
        mesh=mesh,
        scratch_types=[
            pltpu.VMEM((2, K, CHUNK), jnp.int32),
            pltpu.VMEM((rnd_rows, 2 * OUT), jnp.float32),
            pltpu.SemaphoreType.DMA((2,)),
            pltpu.SemaphoreType.DMA,
            pltpu.SemaphoreType.DMA,
        ],
        compiler_params=pltpu.CompilerParams(use_tc_tiling_on_sc=True),
    )
    def gather_kernel(idx_hbm, tab_hbm, out_hbm, idx_v, rows_v, idx_sem,
                      g_sem, o_sem):
        wid = lax.axis_index("s") * 2 + lax.axis_index("c")
        base = wid * rows_per_w
        ibase = row0 // CHUNK + wid * (rows_per_w // CHUNK)

        def idx_src(r):
            return idx_hbm.at[pl.ds(ibase + r * K, K)]

        def out_copy(r):
            return pltpu.make_async_copy(
                rows_v,
                out_hbm.at[pl.ds(base + r * rnd_rows, rnd_rows)],
                o_sem)

        pltpu.async_copy(idx_src(0), idx_v.at[0], idx_sem.at[0])

        def round_body(r, carry):
            s = lax.rem(r, 2)

            @pl.when(r + 1 < rounds)
            def _():
                pltpu.async_copy(idx_src(r + 1), idx_v.at[1 - s],
                                 idx_sem.at[1 - s])

            pltpu.make_async_copy(idx_src(r), idx_v.at[s],
                                  idx_sem.at[s]).wait()

            @pl.when(r > 0)
            def _():
                out_copy(r - 1).wait()

            for j in range(K):
                pltpu.async_copy(tab_hbm.at[idx_v.at[s].at[j]],
                                 rows_v.at[pl.ds(j * CHUNK, CHUNK)], g_sem)
            for j in range(K):
                pltpu.make_async_copy(tab_hbm.at[idx_v.at[s].at[j]],
                                      rows_v.at[pl.ds(j * CHUNK, CHUNK)],
                                      g_sem).wait()

            out_copy(r).start()
            return carry

        lax.fori_loop(0, rounds, round_body, 0)
        out_copy(rounds - 1).wait()

    return gather_kernel(idx2d, table)


def _out_transpose_tc(rows, batch, hist, h0, prev):
    BLKB = 2048
    nb = batch // BLKB
    nh = rows.shape[0] // batch
    out_shape = jax.ShapeDtypeStruct((hist, OUT, batch), jnp.float32)
    in_specs = [
        pl.BlockSpec((BLKB, 2 * OUT), lambda h, j: (h * nb + j, 0)),
    ]
    out_spec = pl.BlockSpec((1, OUT, BLKB), lambda h, j: (h + h0, 0, j))

    if prev is None:
        def body0(r_ref, o_ref):
            o_ref[0] = r_ref[:, :OUT].T

        return pl.pallas_call(
            body0, grid=(nh, nb), in_specs=in_specs, out_specs=out_spec,
            out_shape=out_shape,
        )(rows)

    def body(r_ref, p_ref, o_ref):
        o_ref[0] = r_ref[:, :OUT].T

    return pl.pallas_call(
        body,
        grid=(nh, nb),
        in_specs=in_specs + [pl.BlockSpec((1, 8, 128), lambda h, j: (0, 0, 0))],
        out_specs=out_spec,
        out_shape=out_shape,
        input_output_aliases={1: 0},
    )(rows, prev)


def kernel(X, emb, W, b):
    batch, hist = X.shape
    NCH = 5
    hc = hist // NCH
    idx2d = X.T.reshape(-1, CHUNK).astype(jnp.int32)
    b8 = jnp.broadcast_to(b[None, :], (8, OUT))
    table = _transform_tc(emb.T, W, b8)
    nrows = hc * batch
    p = None
    for i in range(NCH):
        rows = _gather_sc(idx2d, table, nrows, i * nrows)
        p = _out_transpose_tc(rows, batch, hist, i * hc, p)
    return p.transpose(2, 0, 1)


# speedup vs baseline: 2.6717x; 1.1490x over previous
import functools

import jax
import jax.numpy as jnp
from jax import lax
from jax.experimental import pallas as pl
from jax.experimental.pallas import tpu as pltpu
from jax.experimental.pallas import tpu_sc as plsc

HIDDEN = 64
OUT = 64
NW = 32
CHUNK = 128
K = 4


def _transform_tc(embT, W, b8):
    M = embT.shape[1]
    BLK = 8192

    def body(e_ref, w_ref, b_ref, o_ref):
        x = jnp.maximum(e_ref[...], 0.0)
        y = lax.dot_general(x, w_ref[...], (((0,), (1,)), ((), ())),
                            preferred_element_type=jnp.float32)
        y = y + b_ref[0:1, :]
        o_ref[...] = jnp.concatenate([y, y], axis=1)

    return pl.pallas_call(
        body,
        grid=(pl.cdiv(M, BLK),),
        in_specs=[
            pl.BlockSpec((HIDDEN, BLK), lambda i: (0, i)),
            pl.BlockSpec((OUT, HIDDEN), lambda i: (0, 0)),
            pl.BlockSpec((8, OUT), lambda i: (0, 0)),
        ],
        out_specs=pl.BlockSpec((BLK, 2 * OUT), lambda i: (i, 0)),
        out_shape=jax.ShapeDtypeStruct((M, 2 * OUT), jnp.float32),
    )(embT, W, b8)


def _gather_sc(idx2d, table, nrows, row0):
    B = nrows
    rows_per_w = B // NW
    rnd_rows = K * CHUNK
    rounds = rows_per_w // rnd_rows
    mesh = plsc.VectorSubcoreMesh(core_axis_name="c", subcore_axis_name="s")

    @functools.partial(
        pl.kernel,
        out_type=jax.ShapeDtypeStruct((B, 2 * OUT), jnp.float32),
        mesh=mesh,
        scratch_types=[
            pltpu.VMEM((2, K, CHUNK), jnp.int32),
            pltpu.VMEM((rnd_rows, 2 * OUT), jnp.float32),
            pltpu.SemaphoreType.DMA((2,)),
            pltpu.SemaphoreType.DMA,
            pltpu.SemaphoreType.DMA,
        ],
        compiler_params=pltpu.CompilerParams(use_tc_tiling_on_sc=True),
    )
    def gather_kernel(idx_hbm, tab_hbm, out_hbm, idx_v, rows_v, idx_sem,
                      g_sem, o_sem):
        wid = lax.axis_index("s") * 2 + lax.axis_index("c")
        base = wid * rows_per_w
        ibase = row0 // CHUNK + wid * (rows_per_w // CHUNK)

        def idx_src(r):
            return idx_hbm.at[pl.ds(ibase + r * K, K)]

        def out_copy(r):
            return pltpu.make_async_copy(
                rows_v,
                out_hbm.at[pl.ds(base + r * rnd_rows, rnd_rows)],
                o_sem)

        pltpu.async_copy(idx_src(0), idx_v.at[0], idx_sem.at[0])

        def round_body(r, carry):
            s = lax.rem(r, 2)

            @pl.when(r + 1 < rounds)
            def _():
                pltpu.async_copy(idx_src(r + 1), idx_v.at[1 - s],
                                 idx_sem.at[1 - s])

            pltpu.make_async_copy(idx_src(r), idx_v.at[s],
                                  idx_sem.at[s]).wait()

            @pl.when(r > 0)
            def _():
                out_copy(r - 1).wait()

            for j in range(K):
                pltpu.async_copy(tab_hbm.at[idx_v.at[s].at[j]],
                                 rows_v.at[pl.ds(j * CHUNK, CHUNK)], g_sem)
            for j in range(K):
                pltpu.make_async_copy(tab_hbm.at[idx_v.at[s].at[j]],
                                      rows_v.at[pl.ds(j * CHUNK, CHUNK)],
                                      g_sem).wait()

            out_copy(r).start()
            return carry

        lax.fori_loop(0, rounds, round_body, 0)
        out_copy(rounds - 1).wait()

    return gather_kernel(idx2d, table)


def _out_transpose_tc(rows, batch, hist, h0, prev):
    BLKB = 4096
    nb = batch // BLKB
    nh = rows.shape[0] // batch
    out_shape = jax.ShapeDtypeStruct((hist, OUT, batch), jnp.float32)
    in_specs = [
        pl.BlockSpec((BLKB, 2 * OUT), lambda h, j: (h * nb + j, 0)),
    ]
    out_spec = pl.BlockSpec((1, OUT, BLKB), lambda h, j: (h + h0, 0, j))

    if prev is None:
        def body0(r_ref, o_ref):
            o_ref[0] = r_ref[:, :OUT].T

        return pl.pallas_call(
            body0, grid=(nh, nb), in_specs=in_specs, out_specs=out_spec,
            out_shape=out_shape,
        )(rows)

    def body(r_ref, p_ref, o_ref):
        o_ref[0] = r_ref[:, :OUT].T

    return pl.pallas_call(
        body,
        grid=(nh, nb),
        in_specs=in_specs + [pl.BlockSpec((1, 8, 128), lambda h, j: (0, 0, 0))],
        out_specs=out_spec,
        out_shape=out_shape,
        input_output_aliases={1: 0},
    )(rows, prev)


def kernel(X, emb, W, b):
    batch, hist = X.shape
    NCH = 5
    hc = hist // NCH
    idx2d = X.T.reshape(-1, CHUNK).astype(jnp.int32)
    b8 = jnp.broadcast_to(b[None, :], (8, OUT))
    table = _transform_tc(emb.T, W, b8)
    nrows = hc * batch
    p = None
    for i in range(NCH):
        rows = _gather_sc(idx2d, table, nrows, i * nrows)
        p = _out_transpose_tc(rows, batch, hist, i * hc, p)
    return p.transpose(2, 0, 1)


# speedup vs baseline: 2.8114x; 1.0523x over previous
import functools

import jax
import jax.numpy as jnp
from jax import lax
from jax.experimental import pallas as pl
from jax.experimental.pallas import tpu as pltpu
from jax.experimental.pallas import tpu_sc as plsc

HIDDEN = 64
OUT = 64
NW = 32
CHUNK = 128
K = 4


def _transform_tc(embT, W, b8):
    M = embT.shape[1]
    BLK = 16384

    def body(e_ref, w_ref, b_ref, o_ref):
        x = jnp.maximum(e_ref[...], 0.0)
        y = lax.dot_general(x, w_ref[...], (((0,), (1,)), ((), ())),
                            preferred_element_type=jnp.float32)
        y = y + b_ref[0:1, :]
        o_ref[...] = jnp.concatenate([y, y], axis=1)

    return pl.pallas_call(
        body,
        grid=(pl.cdiv(M, BLK),),
        in_specs=[
            pl.BlockSpec((HIDDEN, BLK), lambda i: (0, i)),
            pl.BlockSpec((OUT, HIDDEN), lambda i: (0, 0)),
            pl.BlockSpec((8, OUT), lambda i: (0, 0)),
        ],
        out_specs=pl.BlockSpec((BLK, 2 * OUT), lambda i: (i, 0)),
        out_shape=jax.ShapeDtypeStruct((M, 2 * OUT), jnp.float32),
    )(embT, W, b8)


def _gather_sc(idx2d, table, nrows, row0):
    B = nrows
    rows_per_w = B // NW
    rnd_rows = K * CHUNK
    rounds = rows_per_w // rnd_rows
    mesh = plsc.VectorSubcoreMesh(core_axis_name="c", subcore_axis_name="s")

    @functools.partial(
        pl.kernel,
        out_type=jax.ShapeDtypeStruct((B, 2 * OUT), jnp.float32),
        mesh=mesh,
        scratch_types=[
            pltpu.VMEM((2, K, CHUNK), jnp.int32),
            pltpu.VMEM((rnd_rows, 2 * OUT), jnp.float32),
            pltpu.SemaphoreType.DMA((2,)),
            pltpu.SemaphoreType.DMA,
            pltpu.SemaphoreType.DMA,
        ],
        compiler_params=pltpu.CompilerParams(use_tc_tiling_on_sc=True),
    )
    def gather_kernel(idx_hbm, tab_hbm, out_hbm, idx_v, rows_v, idx_sem,
                      g_sem, o_sem):
        wid = lax.axis_index("s") * 2 + lax.axis_index("c")
        base = wid * rows_per_w
        ibase = row0 // CHUNK + wid * (rows_per_w // CHUNK)

        def idx_src(r):
            return idx_hbm.at[pl.ds(ibase + r * K, K)]

        def out_copy(r):
            return pltpu.make_async_copy(
                rows_v,
                out_hbm.at[pl.ds(base + r * rnd_rows, rnd_rows)],
                o_sem)

        pltpu.async_copy(idx_src(0), idx_v.at[0], idx_sem.at[0])

        def round_body(r, carry):
            s = lax.rem(r, 2)

            @pl.when(r + 1 < rounds)
            def _():
                pltpu.async_copy(idx_src(r + 1), idx_v.at[1 - s],
                                 idx_sem.at[1 - s])

            pltpu.make_async_copy(idx_src(r), idx_v.at[s],
                                  idx_sem.at[s]).wait()

            @pl.when(r > 0)
            def _():
                out_copy(r - 1).wait()

            for j in range(K):
                pltpu.async_copy(tab_hbm.at[idx_v.at[s].at[j]],
                                 rows_v.at[pl.ds(j * CHUNK, CHUNK)], g_sem)
            for j in range(K):
                pltpu.make_async_copy(tab_hbm.at[idx_v.at[s].at[j]],
                                      rows_v.at[pl.ds(j * CHUNK, CHUNK)],
                                      g_sem).wait()

            out_copy(r).start()
            return carry

        lax.fori_loop(0, rounds, round_body, 0)
        out_copy(rounds - 1).wait()

    return gather_kernel(idx2d, table)


def _out_transpose_tc(rows, batch, hist, h0, prev):
    BLKB = 8192
    nb = batch // BLKB
    nh = rows.shape[0] // batch
    out_shape = jax.ShapeDtypeStruct((hist, OUT, batch), jnp.float32)
    in_specs = [
        pl.BlockSpec((BLKB, 2 * OUT), lambda h, j: (h * nb + j, 0)),
    ]
    out_spec = pl.BlockSpec((1, OUT, BLKB), lambda h, j: (h + h0, 0, j))

    if prev is None:
        def body0(r_ref, o_ref):
            o_ref[0] = r_ref[:, :OUT].T

        return pl.pallas_call(
            body0, grid=(nh, nb), in_specs=in_specs, out_specs=out_spec,
            out_shape=out_shape,
        )(rows)

    def body(r_ref, p_ref, o_ref):
        o_ref[0] = r_ref[:, :OUT].T

    return pl.pallas_call(
        body,
        grid=(nh, nb),
        in_specs=in_specs + [pl.BlockSpec((1, 8, 128), lambda h, j: (0, 0, 0))],
        out_specs=out_spec,
        out_shape=out_shape,
        input_output_aliases={1: 0},
    )(rows, prev)


def kernel(X, emb, W, b):
    batch, hist = X.shape
    NCH = 5
    hc = hist // NCH
    idx2d = X.T.reshape(-1, CHUNK).astype(jnp.int32)
    b8 = jnp.broadcast_to(b[None, :], (8, OUT))
    table = _transform_tc(emb.T, W, b8)
    nrows = hc * batch
    p = None
    for i in range(NCH):
        rows = _gather_sc(idx2d, table, nrows, i * nrows)
        p = _out_transpose_tc(rows, batch, hist, i * hc, p)
    return p.transpose(2, 0, 1)
